# baseline (speedup 1.0000x reference)
import functools

import jax
import jax.numpy as jnp
import numpy as np
from jax import lax
from jax.experimental import pallas as pl
from jax.experimental.pallas import tpu as pltpu
from jax.experimental.pallas import tpu_sc as plsc

B = 16384
NBITS = 32
D = 128
G = 4
GB = 8
LUT_ROWS = G * (1 << GB)
LUT_I32 = LUT_ROWS * (D // 2)
NW = 32
RPW = B // NW
CHUNK = 32
SPW = D // 2


def _bit_constants():
    k = np.arange(1 << GB)
    bits = (k[:, None] >> np.arange(GB)[None, :]) & 1
    m0 = np.zeros((LUT_ROWS, NBITS), np.float32)
    m1 = np.zeros((LUT_ROWS, NBITS), np.float32)
    for g in range(G):
        m1[g * 256:(g + 1) * 256, g * GB:(g + 1) * GB] = bits
        m0[g * 256:(g + 1) * 256, g * GB:(g + 1) * GB] = 1 - bits
    return m0, m1


_M0, _M1 = _bit_constants()


def _prep_body(xt_ref, t_ref, m0_ref, m1_ref, gidx_ref, lut_ref):
    for g in range(G):
        acc = xt_ref[g * GB, :] + (g * 256)
        for jb in range(1, GB):
            acc = acc | (xt_ref[g * GB + jb, :] << jb)
        gidx_ref[g, :] = acc
    t0 = t_ref[:, 0, :]
    t1 = t_ref[:, 1, :]
    lutT = (lax.dot_general(t0, m0_ref[...], (((0,), (1,)), ((), ())),
                            preferred_element_type=jnp.float32)
            + lax.dot_general(t1, m1_ref[...], (((0,), (1,)), ((), ())),
                              preferred_element_type=jnp.float32))
    lutT16 = lutT.astype(jnp.bfloat16).reshape(2, D // 2, LUT_ROWS)
    ev = lax.bitcast_convert_type(lutT16[0], jnp.uint16).astype(jnp.uint32)
    od = lax.bitcast_convert_type(lutT16[1], jnp.uint16).astype(jnp.uint32)
    lut_ref[...] = lax.bitcast_convert_type(ev | (od << 16), jnp.int32)


def _prep(x, tables):
    return pl.pallas_call(
        _prep_body,
        out_shape=(jax.ShapeDtypeStruct((G, B), jnp.int32),
                   jax.ShapeDtypeStruct((D // 2, LUT_ROWS), jnp.int32)),
    )(x.T, tables, _M0, _M1)


@functools.lru_cache(maxsize=None)
def _build_sc(interpret=False):
    mesh = plsc.VectorSubcoreMesh(core_axis_name="c", subcore_axis_name="s",
                                  num_cores=2, num_subcores=16)

    @functools.partial(
        pl.kernel, mesh=mesh, interpret=interpret,
        compiler_params=pltpu.CompilerParams(needs_layout_passes=False),
        out_type=jax.ShapeDtypeStruct((B, D), jnp.float32),
        scratch_types=[
            pltpu.VMEM((LUT_I32,), jnp.int32),
            pltpu.VMEM((G, RPW), jnp.int32),
            pltpu.VMEM((16 * SPW,), jnp.int32),
            pltpu.VMEM((2 * CHUNK, D), jnp.float32),
            pltpu.SemaphoreType.DMA,
            pltpu.SemaphoreType.DMA,
        ],
    )
    def sc_lookup(lut_hbm, gidx_hbm, out_hbm, lut_v, gidx_v, pair_v, stage_v,
                  sem0, sem1):
        wid = lax.axis_index("s") * 2 + lax.axis_index("c")
        base = wid * RPW
        pltpu.sync_copy(lut_hbm, lut_v)
        for g in range(G):
            pltpu.sync_copy(gidx_hbm.at[g, pl.ds(base, RPW)], gidx_v.at[g])

        lane = lax.iota(jnp.int32, 16)
        lane16 = lane * 16

        def group_body(j, _):
            buf = (j >> 1) & 1
            half = j & 1

            @pl.when(jnp.logical_and(j >= 4, half == 0))
            def _():
                @pl.when(buf == 0)
                def _():
                    pltpu.make_async_copy(
                        stage_v.at[pl.ds(0, CHUNK), pl.ds(0, D)],
                        out_hbm.at[pl.ds(0, CHUNK)], sem0).wait()

                @pl.when(buf == 1)
                def _():
                    pltpu.make_async_copy(
                        stage_v.at[pl.ds(CHUNK, CHUNK), pl.ds(0, D)],
                        out_hbm.at[pl.ds(0, CHUNK)], sem1).wait()

            r = [gidx_v[g, pl.ds(j * 16, 16)] for g in range(G)]

            @plsc.parallel_loop(0, D // 2, 1, unroll=2)
            def _(p):
                lut_p = lut_v.at[pl.ds(p * LUT_ROWS, LUT_ROWS)]
                v0 = plsc.bitcast(plsc.load_gather(lut_p, [r[0]]), jnp.bfloat16)
                v1 = plsc.bitcast(plsc.load_gather(lut_p, [r[1]]), jnp.bfloat16)
                v2 = plsc.bitcast(plsc.load_gather(lut_p, [r[2]]), jnp.bfloat16)
                v3 = plsc.bitcast(plsc.load_gather(lut_p, [r[3]]), jnp.bfloat16)
                acc = (v0 + v1) + (v2 + v3)
                pair_v[pl.ds(p * 16, 16)] = plsc.bitcast(acc, jnp.int32)

            srow0 = buf * CHUNK + half * 16

            def conv_body(q, _):
                row = q >> 2
                seg = q & 3
                idx = lane16 + (seg * 256 + row)
                pr = plsc.bitcast(plsc.load_gather(pair_v, [idx]), jnp.bfloat16)
                lo, hi = plsc.unpack(pr, format=plsc.PackFormat.INTERLEAVED)
                stage_v[srow0 + row, pl.ds(seg * 16, 16)] = lo
                stage_v[srow0 + row, pl.ds(seg * 16 + D // 2, 16)] = hi
                return 0

            lax.fori_loop(0, 64, conv_body, 0, unroll=2)

            @pl.when(half == 1)
            def _():
                row0 = base + (j >> 1) * CHUNK

                @pl.when(buf == 0)
                def _():
                    pltpu.async_copy(stage_v.at[pl.ds(0, CHUNK), pl.ds(0, D)],
                                     out_hbm.at[pl.ds(row0, CHUNK)], sem0)

                @pl.when(buf == 1)
                def _():
                    pltpu.async_copy(stage_v.at[pl.ds(CHUNK, CHUNK), pl.ds(0, D)],
                                     out_hbm.at[pl.ds(row0, CHUNK)], sem1)

            return 0

        lax.fori_loop(0, RPW // 16, group_body, 0)
        pltpu.make_async_copy(stage_v.at[pl.ds(0, CHUNK), pl.ds(0, D)],
                              out_hbm.at[pl.ds(0, CHUNK)], sem0).wait()
        pltpu.make_async_copy(stage_v.at[pl.ds(CHUNK, CHUNK), pl.ds(0, D)],
                              out_hbm.at[pl.ds(0, CHUNK)], sem1).wait()

    return sc_lookup


def kernel(x, tables):
    gidxs, lut_t = _prep(x, tables)
    return _build_sc()(lut_t.reshape(LUT_I32), gidxs)


# speedup vs baseline: 1.0059x; 1.0059x over previous
import functools

import jax
import jax.numpy as jnp
import numpy as np
from jax import lax
from jax.experimental import pallas as pl
from jax.experimental.pallas import tpu as pltpu
from jax.experimental.pallas import tpu_sc as plsc

B = 16384
NBITS = 32
D = 128
G = 4
GB = 8
LUT_ROWS = G * (1 << GB)
LUT_I32 = LUT_ROWS * (D // 2)
NW = 32
RPW = B // NW
CHUNK = 32
SPW = D // 2


def _bit_constants():
    k = np.arange(1 << GB)
    bits = (k[:, None] >> np.arange(GB)[None, :]) & 1
    m0 = np.zeros((LUT_ROWS, NBITS), np.float32)
    m1 = np.zeros((LUT_ROWS, NBITS), np.float32)
    for g in range(G):
        m1[g * 256:(g + 1) * 256, g * GB:(g + 1) * GB] = bits
        m0[g * 256:(g + 1) * 256, g * GB:(g + 1) * GB] = 1 - bits
    wt = np.zeros((G, NBITS), np.float32)
    for g in range(G):
        wt[g, g * GB:(g + 1) * GB] = 2.0 ** np.arange(GB)
    return m0, m1, wt


_M0, _M1, _WT = _bit_constants()


def _prep_body(xt_ref, t_ref, m0_ref, m1_ref, wt_ref, gidx_ref, lut_ref):
    xf = xt_ref[...].astype(jnp.float32)
    k = lax.dot_general(wt_ref[...], xf, (((1,), (0,)), ((), ())),
                        preferred_element_type=jnp.float32)
    goff = lax.broadcasted_iota(jnp.int32, (G, 1), 0) * 256
    gidx_ref[...] = k.astype(jnp.int32) + goff
    t0 = t_ref[:, 0, :]
    t1 = t_ref[:, 1, :]
    lutT = (lax.dot_general(t0, m0_ref[...], (((0,), (1,)), ((), ())),
                            preferred_element_type=jnp.float32)
            + lax.dot_general(t1, m1_ref[...], (((0,), (1,)), ((), ())),
                              preferred_element_type=jnp.float32))
    lutT16 = lutT.astype(jnp.bfloat16).reshape(2, D // 2, LUT_ROWS)
    ev = lax.bitcast_convert_type(lutT16[0], jnp.uint16).astype(jnp.uint32)
    od = lax.bitcast_convert_type(lutT16[1], jnp.uint16).astype(jnp.uint32)
    lut_ref[...] = lax.bitcast_convert_type(ev | (od << 16), jnp.int32)


def _prep(x, tables):
    return pl.pallas_call(
        _prep_body,
        out_shape=(jax.ShapeDtypeStruct((G, B), jnp.int32),
                   jax.ShapeDtypeStruct((D // 2, LUT_ROWS), jnp.int32)),
    )(x.T, tables, _M0, _M1, _WT)


@functools.lru_cache(maxsize=None)
def _build_sc(interpret=False):
    mesh = plsc.VectorSubcoreMesh(core_axis_name="c", subcore_axis_name="s",
                                  num_cores=2, num_subcores=16)

    @functools.partial(
        pl.kernel, mesh=mesh, interpret=interpret,
        compiler_params=pltpu.CompilerParams(needs_layout_passes=False),
        out_type=jax.ShapeDtypeStruct((B, D), jnp.float32),
        scratch_types=[
            pltpu.VMEM((LUT_I32,), jnp.int32),
            pltpu.VMEM((G, RPW), jnp.int32),
            pltpu.VMEM((16 * SPW,), jnp.int32),
            pltpu.VMEM((2 * CHUNK, D), jnp.float32),
            pltpu.SemaphoreType.DMA,
            pltpu.SemaphoreType.DMA,
        ],
    )
    def sc_lookup(lut_hbm, gidx_hbm, out_hbm, lut_v, gidx_v, pair_v, stage_v,
                  sem0, sem1):
        wid = lax.axis_index("s") * 2 + lax.axis_index("c")
        base = wid * RPW
        pltpu.sync_copy(lut_hbm, lut_v)
        for g in range(G):
            pltpu.sync_copy(gidx_hbm.at[g, pl.ds(base, RPW)], gidx_v.at[g])

        lane = lax.iota(jnp.int32, 16)
        lane16 = lane * 16

        def group_body(j, _):
            buf = (j >> 1) & 1
            half = j & 1

            @pl.when(jnp.logical_and(j >= 4, half == 0))
            def _():
                @pl.when(buf == 0)
                def _():
                    pltpu.make_async_copy(
                        stage_v.at[pl.ds(0, CHUNK), pl.ds(0, D)],
                        out_hbm.at[pl.ds(0, CHUNK)], sem0).wait()

                @pl.when(buf == 1)
                def _():
                    pltpu.make_async_copy(
                        stage_v.at[pl.ds(CHUNK, CHUNK), pl.ds(0, D)],
                        out_hbm.at[pl.ds(0, CHUNK)], sem1).wait()

            r = [gidx_v[g, pl.ds(j * 16, 16)] for g in range(G)]

            @plsc.parallel_loop(0, D // 2, 1, unroll=2)
            def _(p):
                lut_p = lut_v.at[pl.ds(p * LUT_ROWS, LUT_ROWS)]
                v0 = plsc.bitcast(plsc.load_gather(lut_p, [r[0]]), jnp.bfloat16)
                v1 = plsc.bitcast(plsc.load_gather(lut_p, [r[1]]), jnp.bfloat16)
                v2 = plsc.bitcast(plsc.load_gather(lut_p, [r[2]]), jnp.bfloat16)
                v3 = plsc.bitcast(plsc.load_gather(lut_p, [r[3]]), jnp.bfloat16)
                acc = (v0 + v1) + (v2 + v3)
                pair_v[pl.ds(p * 16, 16)] = plsc.bitcast(acc, jnp.int32)

            srow0 = buf * CHUNK + half * 16

            def conv_body(q, _):
                row = q >> 2
                seg = q & 3
                idx = lane16 + (seg * 256 + row)
                pr = plsc.bitcast(plsc.load_gather(pair_v, [idx]), jnp.bfloat16)
                lo, hi = plsc.unpack(pr, format=plsc.PackFormat.INTERLEAVED)
                stage_v[srow0 + row, pl.ds(seg * 16, 16)] = lo
                stage_v[srow0 + row, pl.ds(seg * 16 + D // 2, 16)] = hi
                return 0

            lax.fori_loop(0, 64, conv_body, 0, unroll=2)

            @pl.when(half == 1)
            def _():
                row0 = base + (j >> 1) * CHUNK

                @pl.when(buf == 0)
                def _():
                    pltpu.async_copy(stage_v.at[pl.ds(0, CHUNK), pl.ds(0, D)],
                                     out_hbm.at[pl.ds(row0, CHUNK)], sem0)

                @pl.when(buf == 1)
                def _():
                    pltpu.async_copy(stage_v.at[pl.ds(CHUNK, CHUNK), pl.ds(0, D)],
                                     out_hbm.at[pl.ds(row0, CHUNK)], sem1)

            return 0

        lax.fori_loop(0, RPW // 16, group_body, 0)
        pltpu.make_async_copy(stage_v.at[pl.ds(0, CHUNK), pl.ds(0, D)],
                              out_hbm.at[pl.ds(0, CHUNK)], sem0).wait()
        pltpu.make_async_copy(stage_v.at[pl.ds(CHUNK, CHUNK), pl.ds(0, D)],
                              out_hbm.at[pl.ds(0, CHUNK)], sem1).wait()

    return sc_lookup


def kernel(x, tables):
    gidxs, lut_t = _prep(x, tables)
    return _build_sc()(lut_t.reshape(LUT_I32), gidxs)


# speedup vs baseline: 1.0065x; 1.0006x over previous
import functools

import jax
import jax.numpy as jnp
import numpy as np
from jax import lax
from jax.experimental import pallas as pl
from jax.experimental.pallas import tpu as pltpu
from jax.experimental.pallas import tpu_sc as plsc

B = 16384
NBITS = 32
D = 128
G = 4
GB = 8
LUT_ROWS = G * (1 << GB)
LUT_I32 = LUT_ROWS * (D // 2)
NW = 32
RPW = B // NW
CHUNK = 64
SPW = D // 2


def _bit_constants():
    k = np.arange(1 << GB)
    bits = (k[:, None] >> np.arange(GB)[None, :]) & 1
    m0 = np.zeros((LUT_ROWS, NBITS), np.float32)
    m1 = np.zeros((LUT_ROWS, NBITS), np.float32)
    for g in range(G):
        m1[g * 256:(g + 1) * 256, g * GB:(g + 1) * GB] = bits
        m0[g * 256:(g + 1) * 256, g * GB:(g + 1) * GB] = 1 - bits
    wt = np.zeros((G, NBITS), np.float32)
    for g in range(G):
        wt[g, g * GB:(g + 1) * GB] = 2.0 ** np.arange(GB)
    return m0, m1, wt


_M0, _M1, _WT = _bit_constants()


def _prep_body(xt_ref, t_ref, m0_ref, m1_ref, wt_ref, gidx_ref, lut_ref):
    xf = xt_ref[...].astype(jnp.float32)
    k = lax.dot_general(wt_ref[...], xf, (((1,), (0,)), ((), ())),
                        preferred_element_type=jnp.float32)
    goff = lax.broadcasted_iota(jnp.int32, (G, 1), 0) * 256
    gidx_ref[...] = k.astype(jnp.int32) + goff
    t0 = t_ref[:, 0, :]
    t1 = t_ref[:, 1, :]
    lutT = (lax.dot_general(t0, m0_ref[...], (((0,), (1,)), ((), ())),
                            preferred_element_type=jnp.float32)
            + lax.dot_general(t1, m1_ref[...], (((0,), (1,)), ((), ())),
                              preferred_element_type=jnp.float32))
    lutT16 = lutT.astype(jnp.bfloat16).reshape(2, D // 2, LUT_ROWS)
    ev = lax.bitcast_convert_type(lutT16[0], jnp.uint16).astype(jnp.uint32)
    od = lax.bitcast_convert_type(lutT16[1], jnp.uint16).astype(jnp.uint32)
    lut_ref[...] = lax.bitcast_convert_type(ev | (od << 16), jnp.int32)


def _prep(x, tables):
    return pl.pallas_call(
        _prep_body,
        out_shape=(jax.ShapeDtypeStruct((G, B), jnp.int32),
                   jax.ShapeDtypeStruct((D // 2, LUT_ROWS), jnp.int32)),
    )(x.T, tables, _M0, _M1, _WT)


@functools.lru_cache(maxsize=None)
def _build_sc(interpret=False):
    mesh = plsc.VectorSubcoreMesh(core_axis_name="c", subcore_axis_name="s",
                                  num_cores=2, num_subcores=16)

    @functools.partial(
        pl.kernel, mesh=mesh, interpret=interpret,
        compiler_params=pltpu.CompilerParams(needs_layout_passes=False),
        out_type=jax.ShapeDtypeStruct((B, D), jnp.float32),
        scratch_types=[
            pltpu.VMEM((LUT_I32,), jnp.int32),
            pltpu.VMEM((G, RPW), jnp.int32),
            pltpu.VMEM((16 * SPW,), jnp.int32),
            pltpu.VMEM((2 * CHUNK, D), jnp.float32),
            pltpu.SemaphoreType.DMA,
            pltpu.SemaphoreType.DMA,
        ],
    )
    def sc_lookup(lut_hbm, gidx_hbm, out_hbm, lut_v, gidx_v, pair_v, stage_v,
                  sem0, sem1):
        wid = lax.axis_index("s") * 2 + lax.axis_index("c")
        base = wid * RPW
        pltpu.sync_copy(lut_hbm, lut_v)
        for g in range(G):
            pltpu.sync_copy(gidx_hbm.at[g, pl.ds(base, RPW)], gidx_v.at[g])

        lane = lax.iota(jnp.int32, 16)
        lane16 = lane * 16

        def group_body(j, _):
            buf = (j >> 2) & 1
            half = j & 3

            @pl.when(jnp.logical_and(j >= 8, half == 0))
            def _():
                @pl.when(buf == 0)
                def _():
                    pltpu.make_async_copy(
                        stage_v.at[pl.ds(0, CHUNK), pl.ds(0, D)],
                        out_hbm.at[pl.ds(0, CHUNK)], sem0).wait()

                @pl.when(buf == 1)
                def _():
                    pltpu.make_async_copy(
                        stage_v.at[pl.ds(CHUNK, CHUNK), pl.ds(0, D)],
                        out_hbm.at[pl.ds(0, CHUNK)], sem1).wait()

            r = [gidx_v[g, pl.ds(j * 16, 16)] for g in range(G)]

            @plsc.parallel_loop(0, D // 2, 1, unroll=2)
            def _(p):
                lut_p = lut_v.at[pl.ds(p * LUT_ROWS, LUT_ROWS)]
                v0 = plsc.bitcast(plsc.load_gather(lut_p, [r[0]]), jnp.bfloat16)
                v1 = plsc.bitcast(plsc.load_gather(lut_p, [r[1]]), jnp.bfloat16)
                v2 = plsc.bitcast(plsc.load_gather(lut_p, [r[2]]), jnp.bfloat16)
                v3 = plsc.bitcast(plsc.load_gather(lut_p, [r[3]]), jnp.bfloat16)
                acc = (v0 + v1) + (v2 + v3)
                pair_v[pl.ds(p * 16, 16)] = plsc.bitcast(acc, jnp.int32)

            srow0 = buf * CHUNK + half * 16

            def conv_body(q, _):
                row = q >> 2
                seg = q & 3
                idx = lane16 + (seg * 256 + row)
                pr = plsc.bitcast(plsc.load_gather(pair_v, [idx]), jnp.bfloat16)
                lo, hi = plsc.unpack(pr, format=plsc.PackFormat.INTERLEAVED)
                stage_v[srow0 + row, pl.ds(seg * 16, 16)] = lo
                stage_v[srow0 + row, pl.ds(seg * 16 + D // 2, 16)] = hi
                return 0

            lax.fori_loop(0, 64, conv_body, 0, unroll=2)

            @pl.when(half == 3)
            def _():
                row0 = base + (j >> 2) * CHUNK

                @pl.when(buf == 0)
                def _():
                    pltpu.async_copy(stage_v.at[pl.ds(0, CHUNK), pl.ds(0, D)],
                                     out_hbm.at[pl.ds(row0, CHUNK)], sem0)

                @pl.when(buf == 1)
                def _():
                    pltpu.async_copy(stage_v.at[pl.ds(CHUNK, CHUNK), pl.ds(0, D)],
                                     out_hbm.at[pl.ds(row0, CHUNK)], sem1)

            return 0

        lax.fori_loop(0, RPW // 16, group_body, 0)
        pltpu.make_async_copy(stage_v.at[pl.ds(0, CHUNK), pl.ds(0, D)],
                              out_hbm.at[pl.ds(0, CHUNK)], sem0).wait()
        pltpu.make_async_copy(stage_v.at[pl.ds(CHUNK, CHUNK), pl.ds(0, D)],
                              out_hbm.at[pl.ds(0, CHUNK)], sem1).wait()

    return sc_lookup


def kernel(x, tables):
    gidxs, lut_t = _prep(x, tables)
    return _build_sc()(lut_t.reshape(LUT_I32), gidxs)
